# baseline (speedup 1.0000x reference)
import jax
import jax.numpy as jnp
from jax import lax
from jax.experimental import pallas as pl
from jax.experimental.pallas import tpu as pltpu
from jax.experimental.pallas import tpu_sc as plsc

N_NODES = 10000
N_EDGES = 160000
D_FEAT = 256

NC, NS, L = 2, 16, 16
NW = NC * NS
NPAD = 10240
NODES_PT = NPAD // NS
NODES_HC = NODES_PT // NC

EH = 9984
EXH = N_EDGES - NS * EH
EV = 4992
EXT = 256
EXT0 = NW * EV


def _rowsum_body(x_ref, o_ref):
    o_ref[...] = jnp.dot(x_ref[...], jnp.ones((D_FEAT,), jnp.float32),
                         preferred_element_type=jnp.float32)


def _rsqrt16(d):
    bits = lax.bitcast_convert_type(d, jnp.int32)
    y = lax.bitcast_convert_type(jnp.int32(0x5F3759DF) - (bits >> 1), jnp.float32)
    hd = 0.5 * d
    for _ in range(3):
        y = y * (1.5 - hd * y * y)
    return jnp.where(d > 0.5, y, 0.0)


def _copy_row(src2d, row, dst, n):
    @plsc.parallel_loop(0, n // L, 1, unroll=8)
    def cp(i):
        sl = pl.ds(i * L, L)
        dst[sl] = src2d[row, sl]


def _sc_hist_body(ei, dish,
                  ei2_v, eix_v, toh_v, tox_v, ones_v, deg_v, dis_v, zero_v,
                  hist_sh, sem5, sem6):
    c = lax.axis_index("c")
    s = lax.axis_index("s")
    node0 = s * NODES_PT
    last_s = s == NS - 1

    with jax.named_scope("phA"):
        cph = pltpu.async_copy(ei.at[:, pl.ds(s * EH, EH)], ei2_v, sem5)

        @pl.when(last_s)
        def _():
            cpx = pltpu.async_copy(ei.at[:, pl.ds(NS * EH, EXH)], eix_v, sem6)

        @plsc.parallel_loop(0, EH // L, 1, unroll=4)
        def ob(i):
            ones_v[pl.ds(i * L, L)] = jnp.ones((L,), jnp.float32)

        @plsc.parallel_loop(0, NODES_PT // L, 1, unroll=4)
        def zb(i):
            zero_v[pl.ds(i * L, L)] = jnp.zeros((L,), jnp.float32)
        pltpu.sync_copy(zero_v, hist_sh.at[pl.ds(node0, NODES_PT)])
        cph.wait()
        _copy_row(ei2_v, 1, toh_v, EH)

        @pl.when(last_s)
        def _():
            pltpu.make_async_copy(ei.at[:, pl.ds(NS * EH, EXH)], eix_v, sem6).wait()
            _copy_row(eix_v, 1, tox_v, EXH)
        plsc.subcore_barrier()

    with jax.named_scope("phB"):
        pltpu.sync_copy(ones_v, hist_sh.at[toh_v], add=True)

        @pl.when(last_s)
        def _():
            pltpu.sync_copy(ones_v.at[pl.ds(0, EXH)], hist_sh.at[tox_v], add=True)
        plsc.subcore_barrier()

    with jax.named_scope("phDis"):
        pltpu.sync_copy(hist_sh.at[pl.ds(node0, NODES_PT)], deg_v)

        @plsc.parallel_loop(0, NODES_PT // L, 1, unroll=4)
        def cb(i):
            sl = pl.ds(i * L, L)
            dis_v[sl] = _rsqrt16(deg_v[sl])
        half = c * NODES_HC
        pltpu.sync_copy(dis_v.at[pl.ds(half, NODES_HC)],
                        dish.at[pl.ds(node0 + half, NODES_HC)])


def _sc_prop_body(rowsum, ei, dish, out,
                  ei2_v, eix_v, from_v, to_v, fx_v, tx_v,
                  sf_v, dt_v, outb_v, sfx_v, dtx_v, outx_v,
                  dis_v, s_v, rs_v,
                  dis_sh, s_sh, sem1, sem2, sem3):
    c = lax.axis_index("c")
    s = lax.axis_index("s")
    w = c * NS + s
    node0 = s * NODES_PT
    base = w * EV
    last_w = w == NW - 1

    with jax.named_scope("phC"):
        cpe = pltpu.async_copy(ei.at[:, pl.ds(base, EV)], ei2_v, sem3)
        pltpu.sync_copy(dish.at[pl.ds(node0, NODES_PT)], dis_v)
        pltpu.sync_copy(dis_v, dis_sh.at[pl.ds(node0, NODES_PT)])
        plsc.subcore_barrier()
        cpe.wait()
        _copy_row(ei2_v, 1, to_v, EV)
        cp2 = pltpu.async_copy(dis_sh.at[to_v], dt_v, sem2)
        pltpu.sync_copy(rowsum.at[pl.ds(node0, NODES_PT)], rs_v)

        @plsc.parallel_loop(0, NODES_PT // L, 1, unroll=4)
        def cb(i):
            sl = pl.ds(i * L, L)
            s_v[sl] = dis_v[sl] * rs_v[sl]
        pltpu.sync_copy(s_v, s_sh.at[pl.ds(node0, NODES_PT)])
        _copy_row(ei2_v, 0, from_v, EV)
        plsc.subcore_barrier()

    with jax.named_scope("phD"):
        cp1 = pltpu.async_copy(s_sh.at[from_v], sf_v, sem1)
        cp1.wait()
        cp2.wait()

        @plsc.parallel_loop(0, EV // L, 1, unroll=4)
        def eb(i):
            sl = pl.ds(i * L, L)
            outb_v[sl] = sf_v[sl] * dt_v[sl]
        pltpu.sync_copy(outb_v, out.at[pl.ds(base, EV)])

        @pl.when(last_w)
        def _():
            pltpu.async_copy(ei.at[:, pl.ds(EXT0, EXT)], eix_v, sem3).wait()
            _copy_row(eix_v, 0, fx_v, EXT)
            _copy_row(eix_v, 1, tx_v, EXT)
            cp3 = pltpu.async_copy(s_sh.at[fx_v], sfx_v, sem1)
            cp4 = pltpu.async_copy(dis_sh.at[tx_v], dtx_v, sem2)
            cp3.wait()
            cp4.wait()

            @plsc.parallel_loop(0, EXT // L, 1, unroll=4)
            def ex(i):
                sl = pl.ds(i * L, L)
                outx_v[sl] = sfx_v[sl] * dtx_v[sl]
            pltpu.sync_copy(outx_v, out.at[pl.ds(EXT0, EXT)])


def kernel(x, edge_index):
    x = x.astype(jnp.float32)
    ei = edge_index.astype(jnp.int32)

    mesh = plsc.VectorSubcoreMesh(core_axis_name="c", subcore_axis_name="s")
    sc_hist = pl.kernel(
        _sc_hist_body,
        out_type=jax.ShapeDtypeStruct((NPAD,), jnp.float32),
        mesh=mesh,
        scratch_types=[
            pltpu.VMEM((2, EH), jnp.int32),
            pltpu.VMEM((2, EXH), jnp.int32),
            pltpu.VMEM((EH,), jnp.int32),
            pltpu.VMEM((EXH,), jnp.int32),
            pltpu.VMEM((EH,), jnp.float32),
            pltpu.VMEM((NODES_PT,), jnp.float32),
            pltpu.VMEM((NODES_PT,), jnp.float32),
            pltpu.VMEM((NODES_PT,), jnp.float32),
            pltpu.VMEM_SHARED((NPAD,), jnp.float32),
            pltpu.SemaphoreType.DMA,
            pltpu.SemaphoreType.DMA,
        ],
    )
    dish = sc_hist(ei)

    rowsum1d = pl.pallas_call(
        _rowsum_body,
        grid=(NPAD // 5120,),
        in_specs=[pl.BlockSpec((5120, D_FEAT), lambda i: (i, 0))],
        out_specs=pl.BlockSpec((5120,), lambda i: (i,)),
        out_shape=jax.ShapeDtypeStruct((NPAD,), jnp.float32),
    )(x)

    sc_prop = pl.kernel(
        _sc_prop_body,
        out_type=jax.ShapeDtypeStruct((N_EDGES,), jnp.float32),
        mesh=mesh,
        scratch_types=[
            pltpu.VMEM((2, EV), jnp.int32),
            pltpu.VMEM((2, EXT), jnp.int32),
            pltpu.VMEM((EV,), jnp.int32),
            pltpu.VMEM((EV,), jnp.int32),
            pltpu.VMEM((EXT,), jnp.int32),
            pltpu.VMEM((EXT,), jnp.int32),
            pltpu.VMEM((EV,), jnp.float32),
            pltpu.VMEM((EV,), jnp.float32),
            pltpu.VMEM((EV,), jnp.float32),
            pltpu.VMEM((EXT,), jnp.float32),
            pltpu.VMEM((EXT,), jnp.float32),
            pltpu.VMEM((EXT,), jnp.float32),
            pltpu.VMEM((NODES_PT,), jnp.float32),
            pltpu.VMEM((NODES_PT,), jnp.float32),
            pltpu.VMEM((NODES_PT,), jnp.float32),
            pltpu.VMEM_SHARED((NPAD,), jnp.float32),
            pltpu.VMEM_SHARED((NPAD,), jnp.float32),
            pltpu.SemaphoreType.DMA,
            pltpu.SemaphoreType.DMA,
            pltpu.SemaphoreType.DMA,
        ],
    )
    return sc_prop(rowsum1d, ei, dish)


# speedup vs baseline: 1.0475x; 1.0475x over previous
import jax
import jax.numpy as jnp
from jax import lax
from jax.experimental import pallas as pl
from jax.experimental.pallas import tpu as pltpu
from jax.experimental.pallas import tpu_sc as plsc

N_NODES = 10000
N_EDGES = 160000
D_FEAT = 256

NC, NS, L = 2, 16, 16
NW = NC * NS
NPAD = 10240
NODES_PT = NPAD // NS
NODES_HC = NODES_PT // NC

EHALF = N_EDGES // NC
EH = 4992
EXH = EHALF - NS * EH
EV = 4992
EXT = 256
EXT0 = NW * EV


def _rowsum_body(x_ref, o_ref):
    o_ref[...] = jnp.dot(x_ref[...], jnp.ones((D_FEAT,), jnp.float32),
                         preferred_element_type=jnp.float32)


def _rsqrt16(d):
    bits = lax.bitcast_convert_type(d, jnp.int32)
    y = lax.bitcast_convert_type(jnp.int32(0x5F3759DF) - (bits >> 1), jnp.float32)
    hd = 0.5 * d
    for _ in range(3):
        y = y * (1.5 - hd * y * y)
    return jnp.where(d > 0.5, y, 0.0)


def _copy_row(src2d, row, dst, n):
    @plsc.parallel_loop(0, n // L, 1, unroll=8)
    def cp(i):
        sl = pl.ds(i * L, L)
        dst[sl] = src2d[row, sl]


def _sc_hist_body(ei, degh,
                  ei2_v, eix_v, toh_v, tox_v, ones_v, zero_v,
                  hist_sh, sem5, sem6):
    c = lax.axis_index("c")
    s = lax.axis_index("s")
    node0 = s * NODES_PT
    baseh = c * EHALF + s * EH
    last_s = s == NS - 1

    with jax.named_scope("phA"):
        cph = pltpu.async_copy(ei.at[:, pl.ds(baseh, EH)], ei2_v, sem5)

        @pl.when(last_s)
        def _():
            cpx = pltpu.async_copy(
                ei.at[:, pl.ds(c * EHALF + NS * EH, EXH)], eix_v, sem6)

        @plsc.parallel_loop(0, EH // L, 1, unroll=4)
        def ob(i):
            ones_v[pl.ds(i * L, L)] = jnp.ones((L,), jnp.float32)

        @plsc.parallel_loop(0, NODES_PT // L, 1, unroll=4)
        def zb(i):
            zero_v[pl.ds(i * L, L)] = jnp.zeros((L,), jnp.float32)
        pltpu.sync_copy(zero_v, hist_sh.at[pl.ds(node0, NODES_PT)])
        cph.wait()
        _copy_row(ei2_v, 1, toh_v, EH)

        @pl.when(last_s)
        def _():
            pltpu.make_async_copy(
                ei.at[:, pl.ds(c * EHALF + NS * EH, EXH)], eix_v, sem6).wait()
            _copy_row(eix_v, 1, tox_v, EXH)
        plsc.subcore_barrier()

    with jax.named_scope("phB"):
        pltpu.sync_copy(ones_v, hist_sh.at[toh_v], add=True)

        @pl.when(last_s)
        def _():
            pltpu.sync_copy(ones_v.at[pl.ds(0, EXH)], hist_sh.at[tox_v], add=True)
        plsc.subcore_barrier()

    with jax.named_scope("phW"):
        pltpu.sync_copy(hist_sh.at[pl.ds(node0, NODES_PT)],
                        degh.at[pl.ds(c * NPAD + node0, NODES_PT)])


def _sc_prop_body(rowsum, ei, degh, out,
                  ei2_v, eix_v, from_v, to_v, fx_v, tx_v,
                  sf_v, dt_v, outb_v, sfx_v, dtx_v, outx_v,
                  deg0_v, deg1_v, dis_v, s_v, rs_v,
                  dis_sh, s_sh, sem1, sem2, sem3):
    c = lax.axis_index("c")
    s = lax.axis_index("s")
    w = c * NS + s
    node0 = s * NODES_PT
    base = w * EV
    last_w = w == NW - 1

    with jax.named_scope("phC"):
        cpe = pltpu.async_copy(ei.at[:, pl.ds(base, EV)], ei2_v, sem3)
        pltpu.sync_copy(degh.at[pl.ds(node0, NODES_PT)], deg0_v)
        pltpu.sync_copy(degh.at[pl.ds(NPAD + node0, NODES_PT)], deg1_v)

        @plsc.parallel_loop(0, NODES_PT // L, 1, unroll=4)
        def db(i):
            sl = pl.ds(i * L, L)
            dis_v[sl] = _rsqrt16(deg0_v[sl] + deg1_v[sl])
        pltpu.sync_copy(dis_v, dis_sh.at[pl.ds(node0, NODES_PT)])
        plsc.subcore_barrier()
        cpe.wait()
        _copy_row(ei2_v, 1, to_v, EV)
        cp2 = pltpu.async_copy(dis_sh.at[to_v], dt_v, sem2)
        pltpu.sync_copy(rowsum.at[pl.ds(node0, NODES_PT)], rs_v)

        @plsc.parallel_loop(0, NODES_PT // L, 1, unroll=4)
        def cb(i):
            sl = pl.ds(i * L, L)
            s_v[sl] = dis_v[sl] * rs_v[sl]
        pltpu.sync_copy(s_v, s_sh.at[pl.ds(node0, NODES_PT)])
        _copy_row(ei2_v, 0, from_v, EV)
        plsc.subcore_barrier()

    with jax.named_scope("phD"):
        cp1 = pltpu.async_copy(s_sh.at[from_v], sf_v, sem1)
        cp1.wait()
        cp2.wait()

        @plsc.parallel_loop(0, EV // L, 1, unroll=4)
        def eb(i):
            sl = pl.ds(i * L, L)
            outb_v[sl] = sf_v[sl] * dt_v[sl]
        pltpu.sync_copy(outb_v, out.at[pl.ds(base, EV)])

        @pl.when(last_w)
        def _():
            pltpu.async_copy(ei.at[:, pl.ds(EXT0, EXT)], eix_v, sem3).wait()
            _copy_row(eix_v, 0, fx_v, EXT)
            _copy_row(eix_v, 1, tx_v, EXT)
            cp3 = pltpu.async_copy(s_sh.at[fx_v], sfx_v, sem1)
            cp4 = pltpu.async_copy(dis_sh.at[tx_v], dtx_v, sem2)
            cp3.wait()
            cp4.wait()

            @plsc.parallel_loop(0, EXT // L, 1, unroll=4)
            def ex(i):
                sl = pl.ds(i * L, L)
                outx_v[sl] = sfx_v[sl] * dtx_v[sl]
            pltpu.sync_copy(outx_v, out.at[pl.ds(EXT0, EXT)])


def kernel(x, edge_index):
    x = x.astype(jnp.float32)
    ei = edge_index.astype(jnp.int32)

    mesh = plsc.VectorSubcoreMesh(core_axis_name="c", subcore_axis_name="s")
    sc_hist = pl.kernel(
        _sc_hist_body,
        out_type=jax.ShapeDtypeStruct((NC * NPAD,), jnp.float32),
        mesh=mesh,
        scratch_types=[
            pltpu.VMEM((2, EH), jnp.int32),
            pltpu.VMEM((2, EXH), jnp.int32),
            pltpu.VMEM((EH,), jnp.int32),
            pltpu.VMEM((EXH,), jnp.int32),
            pltpu.VMEM((EH,), jnp.float32),
            pltpu.VMEM((NODES_PT,), jnp.float32),
            pltpu.VMEM_SHARED((NPAD,), jnp.float32),
            pltpu.SemaphoreType.DMA,
            pltpu.SemaphoreType.DMA,
        ],
    )
    degh = sc_hist(ei)

    rowsum1d = pl.pallas_call(
        _rowsum_body,
        grid=(NPAD // 5120,),
        in_specs=[pl.BlockSpec((5120, D_FEAT), lambda i: (i, 0))],
        out_specs=pl.BlockSpec((5120,), lambda i: (i,)),
        out_shape=jax.ShapeDtypeStruct((NPAD,), jnp.float32),
    )(x)

    sc_prop = pl.kernel(
        _sc_prop_body,
        out_type=jax.ShapeDtypeStruct((N_EDGES,), jnp.float32),
        mesh=mesh,
        scratch_types=[
            pltpu.VMEM((2, EV), jnp.int32),
            pltpu.VMEM((2, EXT), jnp.int32),
            pltpu.VMEM((EV,), jnp.int32),
            pltpu.VMEM((EV,), jnp.int32),
            pltpu.VMEM((EXT,), jnp.int32),
            pltpu.VMEM((EXT,), jnp.int32),
            pltpu.VMEM((EV,), jnp.float32),
            pltpu.VMEM((EV,), jnp.float32),
            pltpu.VMEM((EV,), jnp.float32),
            pltpu.VMEM((EXT,), jnp.float32),
            pltpu.VMEM((EXT,), jnp.float32),
            pltpu.VMEM((EXT,), jnp.float32),
            pltpu.VMEM((NODES_PT,), jnp.float32),
            pltpu.VMEM((NODES_PT,), jnp.float32),
            pltpu.VMEM((NODES_PT,), jnp.float32),
            pltpu.VMEM((NODES_PT,), jnp.float32),
            pltpu.VMEM((NODES_PT,), jnp.float32),
            pltpu.VMEM_SHARED((NPAD,), jnp.float32),
            pltpu.VMEM_SHARED((NPAD,), jnp.float32),
            pltpu.SemaphoreType.DMA,
            pltpu.SemaphoreType.DMA,
            pltpu.SemaphoreType.DMA,
        ],
    )
    return sc_prop(rowsum1d, ei, degh)


# speedup vs baseline: 1.0661x; 1.0178x over previous
import jax
import jax.numpy as jnp
from jax import lax
from jax.experimental import pallas as pl
from jax.experimental.pallas import tpu as pltpu
from jax.experimental.pallas import tpu_sc as plsc

N_NODES = 10000
N_EDGES = 160000
D_FEAT = 256

NC, NS, L = 2, 16, 16
NW = NC * NS
NPAD = 10240
NODES_PT = NPAD // NS
NODES_HC = NODES_PT // NC

EHALF = N_EDGES // NC
EH = 4992
EXH = EHALF - NS * EH
EV = 4992
EXT = 256
EXT0 = NW * EV


def _rowsum_body(x_ref, o_ref):
    o_ref[...] = jnp.dot(x_ref[...], jnp.ones((D_FEAT,), jnp.float32),
                         preferred_element_type=jnp.float32)


def _rsqrt16(d):
    bits = lax.bitcast_convert_type(d, jnp.int32)
    y = lax.bitcast_convert_type(jnp.int32(0x5F3759DF) - (bits >> 1), jnp.float32)
    hd = 0.5 * d
    for _ in range(3):
        y = y * (1.5 - hd * y * y)
    return jnp.where(d > 0.5, y, 0.0)


def _copy_row(src2d, row, dst, n):
    @plsc.parallel_loop(0, n // L, 1, unroll=8)
    def cp(i):
        sl = pl.ds(i * L, L)
        dst[sl] = src2d[row, sl]


def _sc_hist_body(ei, degh,
                  ei2_v, eix_v, toh_v, tox_v, ones_v, zero_v,
                  hist_sh, sem5, sem6):
    c = lax.axis_index("c")
    s = lax.axis_index("s")
    node0 = s * NODES_PT
    baseh = c * EHALF + s * EH
    last_s = s == NS - 1

    with jax.named_scope("phA"):
        cph = pltpu.async_copy(ei.at[:, pl.ds(baseh, EH)], ei2_v, sem5)

        @pl.when(last_s)
        def _():
            cpx = pltpu.async_copy(
                ei.at[:, pl.ds(c * EHALF + NS * EH, EXH)], eix_v, sem6)

        @plsc.parallel_loop(0, EH // L, 1, unroll=4)
        def ob(i):
            ones_v[pl.ds(i * L, L)] = jnp.ones((L,), jnp.float32)

        @plsc.parallel_loop(0, NODES_PT // L, 1, unroll=4)
        def zb(i):
            zero_v[pl.ds(i * L, L)] = jnp.zeros((L,), jnp.float32)
        pltpu.sync_copy(zero_v, hist_sh.at[pl.ds(node0, NODES_PT)])
        cph.wait()
        _copy_row(ei2_v, 1, toh_v, EH)

        @pl.when(last_s)
        def _():
            pltpu.make_async_copy(
                ei.at[:, pl.ds(c * EHALF + NS * EH, EXH)], eix_v, sem6).wait()
            _copy_row(eix_v, 1, tox_v, EXH)
        plsc.subcore_barrier()

    with jax.named_scope("phB"):
        pltpu.sync_copy(ones_v, hist_sh.at[toh_v], add=True)

        @pl.when(last_s)
        def _():
            pltpu.sync_copy(ones_v.at[pl.ds(0, EXH)], hist_sh.at[tox_v], add=True)
        plsc.subcore_barrier()

    with jax.named_scope("phW"):
        pltpu.sync_copy(hist_sh.at[pl.ds(node0, NODES_PT)],
                        degh.at[pl.ds(c * NPAD + node0, NODES_PT)])


def _sc_prop_body(rowsum, ei, degh, out,
                  ei2_v, eix_v, from_v, to_v, fx_v, tx_v,
                  sf_v, dt_v, outb_v, sfx_v, dtx_v, outx_v,
                  deg0_v, deg1_v, dis_v, s_v, rs_v,
                  dis_sh, s_sh, sem1, sem2, sem3):
    c = lax.axis_index("c")
    s = lax.axis_index("s")
    w = c * NS + s
    node0 = s * NODES_PT
    base = w * EV
    last_w = w == NW - 1

    with jax.named_scope("phC"):
        cpe = pltpu.async_copy(ei.at[:, pl.ds(base, EV)], ei2_v, sem3)
        cpd = pltpu.async_copy(degh.at[pl.ds(node0, NODES_PT)], deg0_v, sem1)
        pltpu.sync_copy(degh.at[pl.ds(NPAD + node0, NODES_PT)], deg1_v)
        cpd.wait()

        @plsc.parallel_loop(0, NODES_PT // L, 1, unroll=4)
        def db(i):
            sl = pl.ds(i * L, L)
            dis_v[sl] = _rsqrt16(deg0_v[sl] + deg1_v[sl])
        pltpu.sync_copy(dis_v, dis_sh.at[pl.ds(node0, NODES_PT)])
        plsc.subcore_barrier()
        cpe.wait()
        _copy_row(ei2_v, 1, to_v, EV)
        cp2 = pltpu.async_copy(dis_sh.at[to_v], dt_v, sem2)
        pltpu.sync_copy(rowsum.at[pl.ds(node0, NODES_PT)], rs_v)

        @plsc.parallel_loop(0, NODES_PT // L, 1, unroll=4)
        def cb(i):
            sl = pl.ds(i * L, L)
            s_v[sl] = dis_v[sl] * rs_v[sl]
        pltpu.sync_copy(s_v, s_sh.at[pl.ds(node0, NODES_PT)])
        _copy_row(ei2_v, 0, from_v, EV)
        plsc.subcore_barrier()

    with jax.named_scope("phD"):
        EVA = 2560
        EVB = EV - EVA
        cp1a = pltpu.async_copy(s_sh.at[from_v.at[pl.ds(0, EVA)]],
                                sf_v.at[pl.ds(0, EVA)], sem1)
        cp1b = pltpu.async_copy(s_sh.at[from_v.at[pl.ds(EVA, EVB)]],
                                sf_v.at[pl.ds(EVA, EVB)], sem3)
        cp2.wait()
        cp1a.wait()

        @plsc.parallel_loop(0, EVA // L, 1, unroll=4)
        def eba(i):
            sl = pl.ds(i * L, L)
            outb_v[sl] = sf_v[sl] * dt_v[sl]
        cpoa = pltpu.async_copy(outb_v.at[pl.ds(0, EVA)],
                                out.at[pl.ds(base, EVA)], sem2)
        cp1b.wait()

        @plsc.parallel_loop(0, EVB // L, 1, unroll=4)
        def ebb(i):
            sl = pl.ds(EVA + i * L, L)
            outb_v[sl] = sf_v[sl] * dt_v[sl]
        pltpu.sync_copy(outb_v.at[pl.ds(EVA, EVB)], out.at[pl.ds(base + EVA, EVB)])
        cpoa.wait()

        @pl.when(last_w)
        def _():
            pltpu.async_copy(ei.at[:, pl.ds(EXT0, EXT)], eix_v, sem3).wait()
            _copy_row(eix_v, 0, fx_v, EXT)
            _copy_row(eix_v, 1, tx_v, EXT)
            cp3 = pltpu.async_copy(s_sh.at[fx_v], sfx_v, sem1)
            cp4 = pltpu.async_copy(dis_sh.at[tx_v], dtx_v, sem2)
            cp3.wait()
            cp4.wait()

            @plsc.parallel_loop(0, EXT // L, 1, unroll=4)
            def ex(i):
                sl = pl.ds(i * L, L)
                outx_v[sl] = sfx_v[sl] * dtx_v[sl]
            pltpu.sync_copy(outx_v, out.at[pl.ds(EXT0, EXT)])


def kernel(x, edge_index):
    x = x.astype(jnp.float32)
    ei = edge_index.astype(jnp.int32)

    mesh = plsc.VectorSubcoreMesh(core_axis_name="c", subcore_axis_name="s")
    sc_hist = pl.kernel(
        _sc_hist_body,
        out_type=jax.ShapeDtypeStruct((NC * NPAD,), jnp.float32),
        mesh=mesh,
        scratch_types=[
            pltpu.VMEM((2, EH), jnp.int32),
            pltpu.VMEM((2, EXH), jnp.int32),
            pltpu.VMEM((EH,), jnp.int32),
            pltpu.VMEM((EXH,), jnp.int32),
            pltpu.VMEM((EH,), jnp.float32),
            pltpu.VMEM((NODES_PT,), jnp.float32),
            pltpu.VMEM_SHARED((NPAD,), jnp.float32),
            pltpu.SemaphoreType.DMA,
            pltpu.SemaphoreType.DMA,
        ],
    )
    degh = sc_hist(ei)

    rowsum1d = pl.pallas_call(
        _rowsum_body,
        grid=(NPAD // 5120,),
        in_specs=[pl.BlockSpec((5120, D_FEAT), lambda i: (i, 0))],
        out_specs=pl.BlockSpec((5120,), lambda i: (i,)),
        out_shape=jax.ShapeDtypeStruct((NPAD,), jnp.float32),
    )(x)

    sc_prop = pl.kernel(
        _sc_prop_body,
        out_type=jax.ShapeDtypeStruct((N_EDGES,), jnp.float32),
        mesh=mesh,
        scratch_types=[
            pltpu.VMEM((2, EV), jnp.int32),
            pltpu.VMEM((2, EXT), jnp.int32),
            pltpu.VMEM((EV,), jnp.int32),
            pltpu.VMEM((EV,), jnp.int32),
            pltpu.VMEM((EXT,), jnp.int32),
            pltpu.VMEM((EXT,), jnp.int32),
            pltpu.VMEM((EV,), jnp.float32),
            pltpu.VMEM((EV,), jnp.float32),
            pltpu.VMEM((EV,), jnp.float32),
            pltpu.VMEM((EXT,), jnp.float32),
            pltpu.VMEM((EXT,), jnp.float32),
            pltpu.VMEM((EXT,), jnp.float32),
            pltpu.VMEM((NODES_PT,), jnp.float32),
            pltpu.VMEM((NODES_PT,), jnp.float32),
            pltpu.VMEM((NODES_PT,), jnp.float32),
            pltpu.VMEM((NODES_PT,), jnp.float32),
            pltpu.VMEM((NODES_PT,), jnp.float32),
            pltpu.VMEM_SHARED((NPAD,), jnp.float32),
            pltpu.VMEM_SHARED((NPAD,), jnp.float32),
            pltpu.SemaphoreType.DMA,
            pltpu.SemaphoreType.DMA,
            pltpu.SemaphoreType.DMA,
        ],
    )
    return sc_prop(rowsum1d, ei, degh)


# speedup vs baseline: 1.0672x; 1.0010x over previous
import jax
import jax.numpy as jnp
from jax import lax
from jax.experimental import pallas as pl
from jax.experimental.pallas import tpu as pltpu
from jax.experimental.pallas import tpu_sc as plsc

N_NODES = 10000
N_EDGES = 160000
D_FEAT = 256

NC, NS, L = 2, 16, 16
NW = NC * NS
NPAD = 10240
NODES_PT = NPAD // NS
NODES_HC = NODES_PT // NC

EHALF = N_EDGES // NC
EH = 4992
EXH = EHALF - NS * EH
EV = 4992
EXT = 256
EXT0 = NW * EV


def _rowsum_body(x_ref, o_ref):
    o_ref[...] = jnp.dot(x_ref[...], jnp.ones((D_FEAT,), jnp.float32),
                         preferred_element_type=jnp.float32)


def _rsqrt16(d):
    bits = lax.bitcast_convert_type(d, jnp.int32)
    y = lax.bitcast_convert_type(jnp.int32(0x5F3759DF) - (bits >> 1), jnp.float32)
    hd = 0.5 * d
    for _ in range(3):
        y = y * (1.5 - hd * y * y)
    return jnp.where(d > 0.5, y, 0.0)


def _copy_row(src2d, row, dst, n):
    @plsc.parallel_loop(0, n // L, 1, unroll=8)
    def cp(i):
        sl = pl.ds(i * L, L)
        dst[sl] = src2d[row, sl]


def _sc_hist_body(ei, degh,
                  ei2_v, eix_v, toh_v, tox_v, ones_v, zero_v,
                  hist_sh, sem5, sem6):
    c = lax.axis_index("c")
    s = lax.axis_index("s")
    node0 = s * NODES_PT
    baseh = c * EHALF + s * EH
    last_s = s == NS - 1

    with jax.named_scope("phA"):
        cph = pltpu.async_copy(ei.at[:, pl.ds(baseh, EH)], ei2_v, sem5)

        @pl.when(last_s)
        def _():
            cpx = pltpu.async_copy(
                ei.at[:, pl.ds(c * EHALF + NS * EH, EXH)], eix_v, sem6)

        @plsc.parallel_loop(0, EH // L, 1, unroll=4)
        def ob(i):
            ones_v[pl.ds(i * L, L)] = jnp.ones((L,), jnp.float32)

        @plsc.parallel_loop(0, NODES_PT // L, 1, unroll=4)
        def zb(i):
            zero_v[pl.ds(i * L, L)] = jnp.zeros((L,), jnp.float32)
        pltpu.sync_copy(zero_v, hist_sh.at[pl.ds(node0, NODES_PT)])
        cph.wait()
        _copy_row(ei2_v, 1, toh_v, EH)

        @pl.when(last_s)
        def _():
            pltpu.make_async_copy(
                ei.at[:, pl.ds(c * EHALF + NS * EH, EXH)], eix_v, sem6).wait()
            _copy_row(eix_v, 1, tox_v, EXH)
        plsc.subcore_barrier()

    with jax.named_scope("phB"):
        pltpu.sync_copy(ones_v, hist_sh.at[toh_v], add=True)

        @pl.when(last_s)
        def _():
            pltpu.sync_copy(ones_v.at[pl.ds(0, EXH)], hist_sh.at[tox_v], add=True)
        plsc.subcore_barrier()

    with jax.named_scope("phW"):
        pltpu.sync_copy(hist_sh.at[pl.ds(node0, NODES_PT)],
                        degh.at[pl.ds(c * NPAD + node0, NODES_PT)])


def _sc_prop_body(rowsum, ei, degh, out,
                  ei2_v, eix_v, from_v, to_v, fx_v, tx_v,
                  sf_v, dt_v, outb_v, sfx_v, dtx_v, outx_v,
                  deg0_v, deg1_v, dis_v, s_v, rs_v,
                  dis_sh, s_sh, sem1, sem2, sem3, sem4):
    c = lax.axis_index("c")
    s = lax.axis_index("s")
    w = c * NS + s
    node0 = s * NODES_PT
    base = w * EV
    last_w = w == NW - 1

    with jax.named_scope("phC"):
        cpe = pltpu.async_copy(ei.at[:, pl.ds(base, EV)], ei2_v, sem3)
        cpr = pltpu.async_copy(rowsum.at[pl.ds(node0, NODES_PT)], rs_v, sem4)
        cpd = pltpu.async_copy(degh.at[pl.ds(node0, NODES_PT)], deg0_v, sem1)
        pltpu.sync_copy(degh.at[pl.ds(NPAD + node0, NODES_PT)], deg1_v)
        cpd.wait()

        @plsc.parallel_loop(0, NODES_PT // L, 1, unroll=4)
        def db(i):
            sl = pl.ds(i * L, L)
            dis_v[sl] = _rsqrt16(deg0_v[sl] + deg1_v[sl])
        pltpu.sync_copy(dis_v, dis_sh.at[pl.ds(node0, NODES_PT)])
        plsc.subcore_barrier()
        cpe.wait()
        _copy_row(ei2_v, 1, to_v, EV)
        cp2 = pltpu.async_copy(dis_sh.at[to_v], dt_v, sem2)
        _copy_row(ei2_v, 0, from_v, EV)
        cpr.wait()

        @plsc.parallel_loop(0, NODES_PT // L, 1, unroll=4)
        def cb(i):
            sl = pl.ds(i * L, L)
            s_v[sl] = dis_v[sl] * rs_v[sl]
        pltpu.sync_copy(s_v, s_sh.at[pl.ds(node0, NODES_PT)])
        plsc.subcore_barrier()

    with jax.named_scope("phD"):
        EVA = 2560
        EVB = EV - EVA
        cp1a = pltpu.async_copy(s_sh.at[from_v.at[pl.ds(0, EVA)]],
                                sf_v.at[pl.ds(0, EVA)], sem1)
        cp1b = pltpu.async_copy(s_sh.at[from_v.at[pl.ds(EVA, EVB)]],
                                sf_v.at[pl.ds(EVA, EVB)], sem3)
        cp2.wait()
        cp1a.wait()

        @plsc.parallel_loop(0, EVA // L, 1, unroll=4)
        def eba(i):
            sl = pl.ds(i * L, L)
            outb_v[sl] = sf_v[sl] * dt_v[sl]
        cpoa = pltpu.async_copy(outb_v.at[pl.ds(0, EVA)],
                                out.at[pl.ds(base, EVA)], sem2)
        cp1b.wait()

        @plsc.parallel_loop(0, EVB // L, 1, unroll=4)
        def ebb(i):
            sl = pl.ds(EVA + i * L, L)
            outb_v[sl] = sf_v[sl] * dt_v[sl]
        pltpu.sync_copy(outb_v.at[pl.ds(EVA, EVB)], out.at[pl.ds(base + EVA, EVB)])
        cpoa.wait()

        @pl.when(last_w)
        def _():
            pltpu.async_copy(ei.at[:, pl.ds(EXT0, EXT)], eix_v, sem3).wait()
            _copy_row(eix_v, 0, fx_v, EXT)
            _copy_row(eix_v, 1, tx_v, EXT)
            cp3 = pltpu.async_copy(s_sh.at[fx_v], sfx_v, sem1)
            cp4 = pltpu.async_copy(dis_sh.at[tx_v], dtx_v, sem2)
            cp3.wait()
            cp4.wait()

            @plsc.parallel_loop(0, EXT // L, 1, unroll=4)
            def ex(i):
                sl = pl.ds(i * L, L)
                outx_v[sl] = sfx_v[sl] * dtx_v[sl]
            pltpu.sync_copy(outx_v, out.at[pl.ds(EXT0, EXT)])


def kernel(x, edge_index):
    x = x.astype(jnp.float32)
    ei = edge_index.astype(jnp.int32)

    mesh = plsc.VectorSubcoreMesh(core_axis_name="c", subcore_axis_name="s")
    sc_hist = pl.kernel(
        _sc_hist_body,
        out_type=jax.ShapeDtypeStruct((NC * NPAD,), jnp.float32),
        mesh=mesh,
        scratch_types=[
            pltpu.VMEM((2, EH), jnp.int32),
            pltpu.VMEM((2, EXH), jnp.int32),
            pltpu.VMEM((EH,), jnp.int32),
            pltpu.VMEM((EXH,), jnp.int32),
            pltpu.VMEM((EH,), jnp.float32),
            pltpu.VMEM((NODES_PT,), jnp.float32),
            pltpu.VMEM_SHARED((NPAD,), jnp.float32),
            pltpu.SemaphoreType.DMA,
            pltpu.SemaphoreType.DMA,
        ],
    )
    degh = sc_hist(ei)

    rowsum1d = pl.pallas_call(
        _rowsum_body,
        grid=(NPAD // 5120,),
        in_specs=[pl.BlockSpec((5120, D_FEAT), lambda i: (i, 0))],
        out_specs=pl.BlockSpec((5120,), lambda i: (i,)),
        out_shape=jax.ShapeDtypeStruct((NPAD,), jnp.float32),
    )(x)

    sc_prop = pl.kernel(
        _sc_prop_body,
        out_type=jax.ShapeDtypeStruct((N_EDGES,), jnp.float32),
        mesh=mesh,
        scratch_types=[
            pltpu.VMEM((2, EV), jnp.int32),
            pltpu.VMEM((2, EXT), jnp.int32),
            pltpu.VMEM((EV,), jnp.int32),
            pltpu.VMEM((EV,), jnp.int32),
            pltpu.VMEM((EXT,), jnp.int32),
            pltpu.VMEM((EXT,), jnp.int32),
            pltpu.VMEM((EV,), jnp.float32),
            pltpu.VMEM((EV,), jnp.float32),
            pltpu.VMEM((EV,), jnp.float32),
            pltpu.VMEM((EXT,), jnp.float32),
            pltpu.VMEM((EXT,), jnp.float32),
            pltpu.VMEM((EXT,), jnp.float32),
            pltpu.VMEM((NODES_PT,), jnp.float32),
            pltpu.VMEM((NODES_PT,), jnp.float32),
            pltpu.VMEM((NODES_PT,), jnp.float32),
            pltpu.VMEM((NODES_PT,), jnp.float32),
            pltpu.VMEM((NODES_PT,), jnp.float32),
            pltpu.VMEM_SHARED((NPAD,), jnp.float32),
            pltpu.VMEM_SHARED((NPAD,), jnp.float32),
            pltpu.SemaphoreType.DMA,
            pltpu.SemaphoreType.DMA,
            pltpu.SemaphoreType.DMA,
            pltpu.SemaphoreType.DMA,
        ],
    )
    return sc_prop(rowsum1d, ei, degh)


# speedup vs baseline: 1.0676x; 1.0004x over previous
import jax
import jax.numpy as jnp
from jax import lax
from jax.experimental import pallas as pl
from jax.experimental.pallas import tpu as pltpu
from jax.experimental.pallas import tpu_sc as plsc

N_NODES = 10000
N_EDGES = 160000
D_FEAT = 256

NC, NS, L = 2, 16, 16
NW = NC * NS
NPAD = 10240
NODES_PT = NPAD // NS

EHALF = N_EDGES // NC
EH = 4992
EXH = EHALF - NS * EH
EV = 4992
EXT = 256
EXT0 = NW * EV


def _rowsum_body(x_ref, o_ref):
    o_ref[...] = jnp.dot(x_ref[...], jnp.ones((D_FEAT,), jnp.float32),
                         preferred_element_type=jnp.float32)


def _rsqrt16(d):
    bits = lax.bitcast_convert_type(d, jnp.int32)
    y = lax.bitcast_convert_type(jnp.int32(0x5F3759DF) - (bits >> 1), jnp.float32)
    hd = 0.5 * d
    for _ in range(3):
        y = y * (1.5 - hd * y * y)
    return jnp.where(d > 0.5, y, 0.0)


def _copy_row(src2d, row, dst, n):
    @plsc.parallel_loop(0, n // L, 1, unroll=8)
    def cp(i):
        sl = pl.ds(i * L, L)
        dst[sl] = src2d[row, sl]


def _sc_hist_body(ei, degh,
                  ei2_v, eix_v, toh_v, tox_v, ones_v, zero_v,
                  hist_sh, sem5, sem6):
    c = lax.axis_index("c")
    s = lax.axis_index("s")
    node0 = s * NODES_PT
    baseh = c * EHALF + s * EH
    last_s = s == NS - 1

    with jax.named_scope("phA"):
        cph = pltpu.async_copy(ei.at[:, pl.ds(baseh, EH)], ei2_v, sem5)

        @pl.when(last_s)
        def _():
            cpx = pltpu.async_copy(
                ei.at[:, pl.ds(c * EHALF + NS * EH, EXH)], eix_v, sem6)

        @plsc.parallel_loop(0, EH // L, 1, unroll=4)
        def ob(i):
            ones_v[pl.ds(i * L, L)] = jnp.ones((L,), jnp.float32)

        @plsc.parallel_loop(0, NODES_PT // L, 1, unroll=4)
        def zb(i):
            zero_v[pl.ds(i * L, L)] = jnp.zeros((L,), jnp.float32)
        pltpu.sync_copy(zero_v, hist_sh.at[pl.ds(node0, NODES_PT)])
        cph.wait()
        _copy_row(ei2_v, 1, toh_v, EH)

        @pl.when(last_s)
        def _():
            pltpu.make_async_copy(
                ei.at[:, pl.ds(c * EHALF + NS * EH, EXH)], eix_v, sem6).wait()
            _copy_row(eix_v, 1, tox_v, EXH)
        plsc.subcore_barrier()

    with jax.named_scope("phB"):
        pltpu.sync_copy(ones_v, hist_sh.at[toh_v], add=True)

        @pl.when(last_s)
        def _():
            pltpu.sync_copy(ones_v.at[pl.ds(0, EXH)], hist_sh.at[tox_v], add=True)
        plsc.subcore_barrier()

    with jax.named_scope("phW"):
        pltpu.sync_copy(hist_sh.at[pl.ds(node0, NODES_PT)],
                        degh.at[pl.ds(c * NPAD + node0, NODES_PT)])


def _sc_prop_body(rowsum, ei, degh, out,
                  ei2_v, eix_v, from_v, to_v, fx_v, tx_v,
                  sf_v, dt_v, outb_v, sfx_v, dtx_v, outx_v,
                  deg0_v, deg1_v, dis_v, s_v, rs_v,
                  dis_sh, s_sh, sem1, sem2, sem3, sem4):
    c = lax.axis_index("c")
    s = lax.axis_index("s")
    w = c * NS + s
    node0 = s * NODES_PT
    base = w * EV
    last_w = w == NW - 1

    with jax.named_scope("phC"):
        cpe = pltpu.async_copy(ei.at[:, pl.ds(base, EV)], ei2_v, sem3)
        cpr = pltpu.async_copy(rowsum.at[pl.ds(node0, NODES_PT)], rs_v, sem4)
        cpd = pltpu.async_copy(degh.at[pl.ds(node0, NODES_PT)], deg0_v, sem1)
        pltpu.sync_copy(degh.at[pl.ds(NPAD + node0, NODES_PT)], deg1_v)
        cpd.wait()

        @plsc.parallel_loop(0, NODES_PT // L, 1, unroll=4)
        def db(i):
            sl = pl.ds(i * L, L)
            dis_v[sl] = _rsqrt16(deg0_v[sl] + deg1_v[sl])
        pltpu.sync_copy(dis_v, dis_sh.at[pl.ds(node0, NODES_PT)])
        plsc.subcore_barrier()
        cpe.wait()
        _copy_row(ei2_v, 1, to_v, EV)
        cp2 = pltpu.async_copy(dis_sh.at[to_v], dt_v, sem2)
        _copy_row(ei2_v, 0, from_v, EV)
        cpr.wait()

        @plsc.parallel_loop(0, NODES_PT // L, 1, unroll=4)
        def cb(i):
            sl = pl.ds(i * L, L)
            s_v[sl] = dis_v[sl] * rs_v[sl]
        pltpu.sync_copy(s_v, s_sh.at[pl.ds(node0, NODES_PT)])
        plsc.subcore_barrier()

    with jax.named_scope("phD"):
        EVA = 2560
        EVB = EV - EVA
        cp1a = pltpu.async_copy(s_sh.at[from_v.at[pl.ds(0, EVA)]],
                                sf_v.at[pl.ds(0, EVA)], sem1)
        cp1b = pltpu.async_copy(s_sh.at[from_v.at[pl.ds(EVA, EVB)]],
                                sf_v.at[pl.ds(EVA, EVB)], sem3)
        cp2.wait()
        cp1a.wait()

        @plsc.parallel_loop(0, EVA // L, 1, unroll=4)
        def eba(i):
            sl = pl.ds(i * L, L)
            outb_v[sl] = sf_v[sl] * dt_v[sl]
        cpoa = pltpu.async_copy(outb_v.at[pl.ds(0, EVA)],
                                out.at[pl.ds(base, EVA)], sem2)
        cp1b.wait()

        @plsc.parallel_loop(0, EVB // L, 1, unroll=4)
        def ebb(i):
            sl = pl.ds(EVA + i * L, L)
            outb_v[sl] = sf_v[sl] * dt_v[sl]
        pltpu.sync_copy(outb_v.at[pl.ds(EVA, EVB)], out.at[pl.ds(base + EVA, EVB)])
        cpoa.wait()

        @pl.when(last_w)
        def _():
            pltpu.async_copy(ei.at[:, pl.ds(EXT0, EXT)], eix_v, sem3).wait()
            _copy_row(eix_v, 0, fx_v, EXT)
            _copy_row(eix_v, 1, tx_v, EXT)
            cp3 = pltpu.async_copy(s_sh.at[fx_v], sfx_v, sem1)
            cp4 = pltpu.async_copy(dis_sh.at[tx_v], dtx_v, sem2)
            cp3.wait()
            cp4.wait()

            @plsc.parallel_loop(0, EXT // L, 1, unroll=4)
            def ex(i):
                sl = pl.ds(i * L, L)
                outx_v[sl] = sfx_v[sl] * dtx_v[sl]
            pltpu.sync_copy(outx_v, out.at[pl.ds(EXT0, EXT)])


def kernel(x, edge_index):
    x = x.astype(jnp.float32)
    ei = edge_index.astype(jnp.int32)

    mesh = plsc.VectorSubcoreMesh(core_axis_name="c", subcore_axis_name="s")
    sc_hist = pl.kernel(
        _sc_hist_body,
        out_type=jax.ShapeDtypeStruct((NC * NPAD,), jnp.float32),
        mesh=mesh,
        scratch_types=[
            pltpu.VMEM((2, EH), jnp.int32),
            pltpu.VMEM((2, EXH), jnp.int32),
            pltpu.VMEM((EH,), jnp.int32),
            pltpu.VMEM((EXH,), jnp.int32),
            pltpu.VMEM((EH,), jnp.float32),
            pltpu.VMEM((NODES_PT,), jnp.float32),
            pltpu.VMEM_SHARED((NPAD,), jnp.float32),
            pltpu.SemaphoreType.DMA,
            pltpu.SemaphoreType.DMA,
        ],
    )
    degh = sc_hist(ei)

    rowsum1d = pl.pallas_call(
        _rowsum_body,
        grid=(NPAD // 5120,),
        in_specs=[pl.BlockSpec((5120, D_FEAT), lambda i: (i, 0))],
        out_specs=pl.BlockSpec((5120,), lambda i: (i,)),
        out_shape=jax.ShapeDtypeStruct((NPAD,), jnp.float32),
    )(x)

    sc_prop = pl.kernel(
        _sc_prop_body,
        out_type=jax.ShapeDtypeStruct((N_EDGES,), jnp.float32),
        mesh=mesh,
        scratch_types=[
            pltpu.VMEM((2, EV), jnp.int32),
            pltpu.VMEM((2, EXT), jnp.int32),
            pltpu.VMEM((EV,), jnp.int32),
            pltpu.VMEM((EV,), jnp.int32),
            pltpu.VMEM((EXT,), jnp.int32),
            pltpu.VMEM((EXT,), jnp.int32),
            pltpu.VMEM((EV,), jnp.float32),
            pltpu.VMEM((EV,), jnp.float32),
            pltpu.VMEM((EV,), jnp.float32),
            pltpu.VMEM((EXT,), jnp.float32),
            pltpu.VMEM((EXT,), jnp.float32),
            pltpu.VMEM((EXT,), jnp.float32),
            pltpu.VMEM((NODES_PT,), jnp.float32),
            pltpu.VMEM((NODES_PT,), jnp.float32),
            pltpu.VMEM((NODES_PT,), jnp.float32),
            pltpu.VMEM((NODES_PT,), jnp.float32),
            pltpu.VMEM((NODES_PT,), jnp.float32),
            pltpu.VMEM_SHARED((NPAD,), jnp.float32),
            pltpu.VMEM_SHARED((NPAD,), jnp.float32),
            pltpu.SemaphoreType.DMA,
            pltpu.SemaphoreType.DMA,
            pltpu.SemaphoreType.DMA,
            pltpu.SemaphoreType.DMA,
        ],
    )
    return sc_prop(rowsum1d, ei, degh)


# speedup vs baseline: 1.0702x; 1.0024x over previous
import jax
import jax.numpy as jnp
from jax import lax
from jax.experimental import pallas as pl
from jax.experimental.pallas import tpu as pltpu
from jax.experimental.pallas import tpu_sc as plsc

N_NODES = 10000
N_EDGES = 160000
D_FEAT = 256

NC, NS, L = 2, 16, 16
NW = NC * NS
NPAD = 10240
NODES_PT = NPAD // NS

EHALF = N_EDGES // NC
EH = 4992
EXH = EHALF - NS * EH
EV = 4992
EXT = 256
EXT0 = NW * EV


def _rowsum_body(x_ref, o_ref):
    o_ref[...] = jnp.dot(x_ref[...], jnp.ones((D_FEAT,), jnp.float32),
                         preferred_element_type=jnp.float32)


def _rsqrt16(d):
    bits = lax.bitcast_convert_type(d, jnp.int32)
    y = lax.bitcast_convert_type(jnp.int32(0x5F3759DF) - (bits >> 1), jnp.float32)
    hd = 0.5 * d
    for _ in range(3):
        y = y * (1.5 - hd * y * y)
    return jnp.where(d > 0.5, y, 0.0)


def _copy_row(src2d, row, dst, n):
    @plsc.parallel_loop(0, n // L, 1, unroll=8)
    def cp(i):
        sl = pl.ds(i * L, L)
        dst[sl] = src2d[row, sl]


def _sc_hist_body(ei, degh,
                  ei2_v, eix_v, toh_v, tox_v, ones_v, zero_v,
                  hist_sh, sem5, sem6):
    c = lax.axis_index("c")
    s = lax.axis_index("s")
    node0 = s * NODES_PT
    baseh = c * EHALF + s * EH
    last_s = s == NS - 1

    with jax.named_scope("phA"):
        cph = pltpu.async_copy(ei.at[:, pl.ds(baseh, EH)], ei2_v, sem5)

        @pl.when(last_s)
        def _():
            cpx = pltpu.async_copy(
                ei.at[:, pl.ds(c * EHALF + NS * EH, EXH)], eix_v, sem6)

        @plsc.parallel_loop(0, EH // L, 1, unroll=4)
        def ob(i):
            ones_v[pl.ds(i * L, L)] = jnp.ones((L,), jnp.float32)

        @plsc.parallel_loop(0, NODES_PT // L, 1, unroll=4)
        def zb(i):
            zero_v[pl.ds(i * L, L)] = jnp.zeros((L,), jnp.float32)
        pltpu.sync_copy(zero_v, hist_sh.at[pl.ds(node0, NODES_PT)])
        cph.wait()
        _copy_row(ei2_v, 1, toh_v, EH)

        @pl.when(last_s)
        def _():
            pltpu.make_async_copy(
                ei.at[:, pl.ds(c * EHALF + NS * EH, EXH)], eix_v, sem6).wait()
            _copy_row(eix_v, 1, tox_v, EXH)
        plsc.subcore_barrier()

    with jax.named_scope("phB"):
        pltpu.sync_copy(ones_v, hist_sh.at[toh_v], add=True)

        @pl.when(last_s)
        def _():
            pltpu.sync_copy(ones_v.at[pl.ds(0, EXH)], hist_sh.at[tox_v], add=True)
        plsc.subcore_barrier()

    with jax.named_scope("phW"):
        pltpu.sync_copy(hist_sh.at[pl.ds(node0, NODES_PT)],
                        degh.at[pl.ds(c * NPAD + node0, NODES_PT)])


def _sc_prop_body(rowsum, ei, degh, out,
                  ei2_v, eix_v, from_v, to_v, fx_v, tx_v,
                  sf_v, dt_v, outb_v, sfx_v, dtx_v, outx_v,
                  deg0_v, deg1_v, dis_v, s_v, rs_v,
                  dis_sh, s_sh, sem1, sem2, sem3, sem4, sem5):
    c = lax.axis_index("c")
    s = lax.axis_index("s")
    w = c * NS + s
    node0 = s * NODES_PT
    base = w * EV
    last_w = w == NW - 1

    with jax.named_scope("phC"):
        cpe = pltpu.async_copy(ei.at[:, pl.ds(base, EV)], ei2_v, sem3)
        cpr = pltpu.async_copy(rowsum.at[pl.ds(node0, NODES_PT)], rs_v, sem4)

        @pl.when(last_w)
        def _():
            cpx = pltpu.async_copy(ei.at[:, pl.ds(EXT0, EXT)], eix_v, sem5)
        cpd = pltpu.async_copy(degh.at[pl.ds(node0, NODES_PT)], deg0_v, sem1)
        pltpu.sync_copy(degh.at[pl.ds(NPAD + node0, NODES_PT)], deg1_v)
        cpd.wait()

        @plsc.parallel_loop(0, NODES_PT // L, 1, unroll=4)
        def db(i):
            sl = pl.ds(i * L, L)
            dis_v[sl] = _rsqrt16(deg0_v[sl] + deg1_v[sl])
        pltpu.sync_copy(dis_v, dis_sh.at[pl.ds(node0, NODES_PT)])
        plsc.subcore_barrier()
        cpe.wait()
        _copy_row(ei2_v, 1, to_v, EV)
        cp2 = pltpu.async_copy(dis_sh.at[to_v], dt_v, sem2)
        _copy_row(ei2_v, 0, from_v, EV)
        cpr.wait()

        @plsc.parallel_loop(0, NODES_PT // L, 1, unroll=4)
        def cb(i):
            sl = pl.ds(i * L, L)
            s_v[sl] = dis_v[sl] * rs_v[sl]
        pltpu.sync_copy(s_v, s_sh.at[pl.ds(node0, NODES_PT)])

        @pl.when(last_w)
        def _():
            pltpu.make_async_copy(ei.at[:, pl.ds(EXT0, EXT)], eix_v, sem5).wait()
            _copy_row(eix_v, 0, fx_v, EXT)
            _copy_row(eix_v, 1, tx_v, EXT)
        plsc.subcore_barrier()

    with jax.named_scope("phD"):
        EVA = 2560
        EVB = EV - EVA
        cp1a = pltpu.async_copy(s_sh.at[from_v.at[pl.ds(0, EVA)]],
                                sf_v.at[pl.ds(0, EVA)], sem1)
        cp1b = pltpu.async_copy(s_sh.at[from_v.at[pl.ds(EVA, EVB)]],
                                sf_v.at[pl.ds(EVA, EVB)], sem3)

        @pl.when(last_w)
        def _():
            cp3 = pltpu.async_copy(s_sh.at[fx_v], sfx_v, sem5)
            cp4 = pltpu.async_copy(dis_sh.at[tx_v], dtx_v, sem4)
        cp2.wait()
        cp1a.wait()

        @plsc.parallel_loop(0, EVA // L, 1, unroll=4)
        def eba(i):
            sl = pl.ds(i * L, L)
            outb_v[sl] = sf_v[sl] * dt_v[sl]
        cpoa = pltpu.async_copy(outb_v.at[pl.ds(0, EVA)],
                                out.at[pl.ds(base, EVA)], sem2)
        cp1b.wait()

        @plsc.parallel_loop(0, EVB // L, 1, unroll=4)
        def ebb(i):
            sl = pl.ds(EVA + i * L, L)
            outb_v[sl] = sf_v[sl] * dt_v[sl]
        pltpu.sync_copy(outb_v.at[pl.ds(EVA, EVB)], out.at[pl.ds(base + EVA, EVB)])
        cpoa.wait()

        @pl.when(last_w)
        def _():
            pltpu.make_async_copy(s_sh.at[fx_v], sfx_v, sem5).wait()
            pltpu.make_async_copy(dis_sh.at[tx_v], dtx_v, sem4).wait()

            @plsc.parallel_loop(0, EXT // L, 1, unroll=4)
            def ex(i):
                sl = pl.ds(i * L, L)
                outx_v[sl] = sfx_v[sl] * dtx_v[sl]
            pltpu.sync_copy(outx_v, out.at[pl.ds(EXT0, EXT)])


def kernel(x, edge_index):
    x = x.astype(jnp.float32)
    ei = edge_index.astype(jnp.int32)

    mesh = plsc.VectorSubcoreMesh(core_axis_name="c", subcore_axis_name="s")
    sc_hist = pl.kernel(
        _sc_hist_body,
        out_type=jax.ShapeDtypeStruct((NC * NPAD,), jnp.float32),
        mesh=mesh,
        scratch_types=[
            pltpu.VMEM((2, EH), jnp.int32),
            pltpu.VMEM((2, EXH), jnp.int32),
            pltpu.VMEM((EH,), jnp.int32),
            pltpu.VMEM((EXH,), jnp.int32),
            pltpu.VMEM((EH,), jnp.float32),
            pltpu.VMEM((NODES_PT,), jnp.float32),
            pltpu.VMEM_SHARED((NPAD,), jnp.float32),
            pltpu.SemaphoreType.DMA,
            pltpu.SemaphoreType.DMA,
        ],
    )
    degh = sc_hist(ei)

    rowsum1d = pl.pallas_call(
        _rowsum_body,
        grid=(NPAD // 5120,),
        in_specs=[pl.BlockSpec((5120, D_FEAT), lambda i: (i, 0))],
        out_specs=pl.BlockSpec((5120,), lambda i: (i,)),
        out_shape=jax.ShapeDtypeStruct((NPAD,), jnp.float32),
    )(x)

    sc_prop = pl.kernel(
        _sc_prop_body,
        out_type=jax.ShapeDtypeStruct((N_EDGES,), jnp.float32),
        mesh=mesh,
        scratch_types=[
            pltpu.VMEM((2, EV), jnp.int32),
            pltpu.VMEM((2, EXT), jnp.int32),
            pltpu.VMEM((EV,), jnp.int32),
            pltpu.VMEM((EV,), jnp.int32),
            pltpu.VMEM((EXT,), jnp.int32),
            pltpu.VMEM((EXT,), jnp.int32),
            pltpu.VMEM((EV,), jnp.float32),
            pltpu.VMEM((EV,), jnp.float32),
            pltpu.VMEM((EV,), jnp.float32),
            pltpu.VMEM((EXT,), jnp.float32),
            pltpu.VMEM((EXT,), jnp.float32),
            pltpu.VMEM((EXT,), jnp.float32),
            pltpu.VMEM((NODES_PT,), jnp.float32),
            pltpu.VMEM((NODES_PT,), jnp.float32),
            pltpu.VMEM((NODES_PT,), jnp.float32),
            pltpu.VMEM((NODES_PT,), jnp.float32),
            pltpu.VMEM((NODES_PT,), jnp.float32),
            pltpu.VMEM_SHARED((NPAD,), jnp.float32),
            pltpu.VMEM_SHARED((NPAD,), jnp.float32),
            pltpu.SemaphoreType.DMA,
            pltpu.SemaphoreType.DMA,
            pltpu.SemaphoreType.DMA,
            pltpu.SemaphoreType.DMA,
            pltpu.SemaphoreType.DMA,
        ],
    )
    return sc_prop(rowsum1d, ei, degh)
